# Initial kernel scaffold; baseline (speedup 1.0000x reference)
import jax
import jax.numpy as jnp
from jax.experimental import pallas as pl


def kernel(x, table):
    raise NotImplementedError("write your pallas kernel here")


# speedup vs baseline: 6.2041x; 6.2041x over previous
import functools
import math

import jax
import jax.numpy as jnp
from jax import lax
from jax.experimental import pallas as pl
from jax.experimental.pallas import tpu as pltpu
from jax.experimental.pallas import tpu_sc as plsc

D_MODEL = 128
SCALE = math.sqrt(float(D_MODEL))
LANES = 16
NUM_WORKERS = 32
CHUNK = 512


def _emb_body(x_hbm, table_hbm, out_hbm, idx_v, rows_v, sem, *, bpw, n_chunks):
    wid = lax.axis_index("s") * 2 + lax.axis_index("c")
    base = wid * bpw

    def chunk_body(ci, carry):
        off = base + ci * CHUNK
        pltpu.sync_copy(x_hbm.at[pl.ds(off, CHUNK)], idx_v)
        pltpu.async_copy(table_hbm.at[idx_v], rows_v, sem).wait()

        def scale_row(r, c):
            for k in range(D_MODEL // LANES):
                sl = pl.ds(k * LANES, LANES)
                rows_v[r, sl] = rows_v[r, sl] * SCALE
            return c

        lax.fori_loop(0, CHUNK, scale_row, 0)
        pltpu.sync_copy(rows_v, out_hbm.at[pl.ds(off, CHUNK)])
        return carry

    lax.fori_loop(0, n_chunks, chunk_body, 0)


@functools.partial(jax.jit, static_argnames=())
def kernel(x, table):
    b, h = x.shape
    n = b * h
    x_flat = x.reshape(n).astype(jnp.int32)
    bpw = n // NUM_WORKERS
    n_chunks = bpw // CHUNK

    mesh = plsc.VectorSubcoreMesh(core_axis_name="c", subcore_axis_name="s")
    grid_kernel = pl.kernel(
        functools.partial(_emb_body, bpw=bpw, n_chunks=n_chunks),
        out_type=jax.ShapeDtypeStruct((n, D_MODEL), jnp.float32),
        mesh=mesh,
        scratch_types=[
            pltpu.VMEM((CHUNK,), jnp.int32),
            pltpu.VMEM((CHUNK, D_MODEL), jnp.float32),
            pltpu.SemaphoreType.DMA,
        ],
    )
    out = grid_kernel(x_flat, table)
    return out.reshape(b, h, D_MODEL)


# speedup vs baseline: 9.1393x; 1.4731x over previous
import functools
import math

import jax
import jax.numpy as jnp
from jax import lax
from jax.experimental import pallas as pl
from jax.experimental.pallas import tpu as pltpu
from jax.experimental.pallas import tpu_sc as plsc

D_MODEL = 128
SCALE = math.sqrt(float(D_MODEL))
LANES = 16
NUM_WORKERS = 32
CHUNK = 320


def _emb_body(x_hbm, table_hbm, out_hbm, idx_all, rows0, rows1, sg0, sg1,
              sw0, sw1, *, bpw, n_chunks):
    rows = (rows0, rows1)
    sg = (sg0, sg1)
    sw = (sw0, sw1)
    wid = lax.axis_index("s") * 2 + lax.axis_index("c")
    base = wid * bpw

    pltpu.sync_copy(x_hbm.at[pl.ds(base, bpw)], idx_all)

    def start_gather(ci, b):
        pltpu.async_copy(table_hbm.at[idx_all.at[pl.ds(ci * CHUNK, CHUNK)]],
                         rows[b], sg[b])

    def wait_gather(b):
        pltpu.make_async_copy(table_hbm.at[idx_all.at[pl.ds(0, CHUNK)]],
                              rows[b], sg[b]).wait()

    def start_writeback(ci, b):
        pltpu.async_copy(rows[b], out_hbm.at[pl.ds(base + ci * CHUNK, CHUNK)],
                         sw[b])

    def wait_writeback(b):
        pltpu.make_async_copy(rows[b], out_hbm.at[pl.ds(base, CHUNK)],
                              sw[b]).wait()

    start_gather(0, 0)

    def outer(g, carry):
        for b in (0, 1):
            ci = 2 * g + b
            nb = 1 - b
            wait_gather(b)

            @pl.when(ci + 1 < n_chunks)
            def _():
                @pl.when(ci >= 1)
                def _():
                    wait_writeback(nb)
                start_gather(ci + 1, nb)

            def scale_row(r, c):
                for k in range(D_MODEL // LANES):
                    sl = pl.ds(k * LANES, LANES)
                    rows[b][r, sl] = rows[b][r, sl] * SCALE
                return c

            lax.fori_loop(0, CHUNK, scale_row, 0)
            start_writeback(ci, b)
        return carry

    lax.fori_loop(0, n_chunks // 2, outer, 0)
    wait_writeback(0)
    wait_writeback(1)


@functools.partial(jax.jit, static_argnames=())
def kernel(x, table):
    b, h = x.shape
    n = b * h
    x_flat = x.reshape(n).astype(jnp.int32)
    bpw = n // NUM_WORKERS
    n_chunks = bpw // CHUNK

    mesh = plsc.VectorSubcoreMesh(core_axis_name="c", subcore_axis_name="s")
    grid_kernel = pl.kernel(
        functools.partial(_emb_body, bpw=bpw, n_chunks=n_chunks),
        out_type=jax.ShapeDtypeStruct((n, D_MODEL), jnp.float32),
        mesh=mesh,
        scratch_types=[
            pltpu.VMEM((bpw,), jnp.int32),
            pltpu.VMEM((CHUNK, D_MODEL), jnp.float32),
            pltpu.VMEM((CHUNK, D_MODEL), jnp.float32),
            pltpu.SemaphoreType.DMA,
            pltpu.SemaphoreType.DMA,
            pltpu.SemaphoreType.DMA,
            pltpu.SemaphoreType.DMA,
        ],
    )
    out = grid_kernel(x_flat, table)
    return out.reshape(b, h, D_MODEL)
